# Initial kernel scaffold; baseline (speedup 1.0000x reference)
import jax
import jax.numpy as jnp
from jax.experimental import pallas as pl


def kernel(spikes, neuron_regions, eids, enc_w, enc_b, dec_w, dec_b):
    raise NotImplementedError("write your pallas kernel here")


# speedup vs baseline: 5.6881x; 5.6881x over previous
import functools

import jax
import jax.numpy as jnp
from jax.experimental import pallas as pl


def _codec_body(segs, sp_ref, e_ref, d_ref, eb_ref, db_ref, out_ref):
    for i, (off, n_r) in enumerate(segs):
        sp_r = sp_ref[:, off:off + n_r]
        e_r = e_ref[:, off:off + n_r]
        tok = jax.lax.dot_general(
            sp_r, e_r, (((1,), (1,)), ((), ())),
            preferred_element_type=jnp.float32)
        tok = tok + eb_ref[i:i + 1, :]
        d_r = d_ref[off:off + n_r, :]
        rec = jax.lax.dot_general(
            tok, d_r, (((1,), (1,)), ((), ())),
            preferred_element_type=jnp.float32)
        out_ref[:, off:off + n_r] = rec + db_ref[:, off:off + n_r]


def kernel(spikes, neuron_regions, eids, enc_w, enc_b, dec_w, dec_b):
    B, T, N = spikes.shape
    M = B * T
    D = enc_w[0].shape[0]
    sizes = [w.shape[1] for w in enc_w]
    offs = [0]
    for n in sizes:
        offs.append(offs[-1] + n)
    segs = tuple((offs[i], sizes[i]) for i in range(len(sizes)))

    sp2 = spikes.reshape(M, N)
    E = jnp.concatenate(enc_w, axis=1)
    Dc = jnp.concatenate(dec_w, axis=0)
    EB = jnp.stack(enc_b, axis=0)
    DB = jnp.concatenate(dec_b)[None, :]

    TM = 160
    grid = (M // TM,)
    out = pl.pallas_call(
        functools.partial(_codec_body, segs),
        grid=grid,
        in_specs=[
            pl.BlockSpec((TM, N), lambda i: (i, 0)),
            pl.BlockSpec((D, N), lambda i: (0, 0)),
            pl.BlockSpec((N, D), lambda i: (0, 0)),
            pl.BlockSpec(EB.shape, lambda i: (0, 0)),
            pl.BlockSpec((1, N), lambda i: (0, 0)),
        ],
        out_specs=pl.BlockSpec((TM, N), lambda i: (i, 0)),
        out_shape=jax.ShapeDtypeStruct((M, N), spikes.dtype),
    )(sp2, E, Dc, EB, DB)
    return out.reshape(B, T, N)
